# baseline (speedup 1.0000x reference)
import functools

import jax
import jax.numpy as jnp
from jax import lax
from jax.experimental import pallas as pl
from jax.experimental.pallas import tpu as pltpu
from jax.experimental.pallas import tpu_sc as plsc


def kernel(ctx, prefix, suffix):
    n_ctx, d = ctx.shape
    n_cls = prefix.shape[0]
    n_suf = suffix.shape[1]
    seq = 1 + n_ctx + n_suf
    row = seq * d
    ctx_sz = n_ctx * d
    suf_sz = n_suf * d

    info = plsc.get_sparse_core_info()
    nw = info.num_cores * info.num_subcores
    cpw = (n_cls + nw - 1) // nw

    mesh = plsc.VectorSubcoreMesh(core_axis_name="c", subcore_axis_name="s")

    @functools.partial(
        pl.kernel,
        out_type=jax.ShapeDtypeStruct((n_cls * row,), jnp.float32),
        mesh=mesh,
    )
    def body(ctx_hbm, prefix_hbm, suffix_hbm, out_hbm):
        wid = lax.axis_index("s") * info.num_cores + lax.axis_index("c")
        base = wid * cpw
        for j in range(cpw):
            i = base + j
            o = i * row

            @pl.when(i < n_cls)
            def _():
                pltpu.sync_copy(
                    prefix_hbm.at[pl.ds(i * d, d)], out_hbm.at[pl.ds(o, d)]
                )
                pltpu.sync_copy(ctx_hbm, out_hbm.at[pl.ds(o + d, ctx_sz)])
                pltpu.sync_copy(
                    suffix_hbm.at[pl.ds(i * suf_sz, suf_sz)],
                    out_hbm.at[pl.ds(o + d + ctx_sz, suf_sz)],
                )

    flat = body(
        ctx.reshape(-1), prefix.reshape(-1), suffix.reshape(-1)
    )
    return flat.reshape(n_cls, seq, d)


# speedup vs baseline: 4.0705x; 4.0705x over previous
import functools

import jax
import jax.numpy as jnp
from jax import lax
from jax.experimental import pallas as pl
from jax.experimental.pallas import tpu as pltpu
from jax.experimental.pallas import tpu_sc as plsc


def kernel(ctx, prefix, suffix):
    n_ctx, d = ctx.shape
    n_cls = prefix.shape[0]
    n_suf = suffix.shape[1]
    seq = 1 + n_ctx + n_suf
    row = seq * d
    ctx_sz = n_ctx * d
    suf_sz = n_suf * d

    info = plsc.get_sparse_core_info()
    nw = info.num_cores * info.num_subcores
    cpw = (n_cls + nw - 1) // nw

    mesh = plsc.VectorSubcoreMesh(core_axis_name="c", subcore_axis_name="s")

    @functools.partial(
        pl.kernel,
        out_type=jax.ShapeDtypeStruct((n_cls * row,), jnp.float32),
        mesh=mesh,
        scratch_types=[
            pltpu.VMEM((row,), jnp.float32),
            pltpu.VMEM((row,), jnp.float32),
            pltpu.SemaphoreType.DMA,
            pltpu.SemaphoreType.DMA,
            pltpu.SemaphoreType.DMA,
            pltpu.SemaphoreType.DMA,
        ],
    )
    def body(ctx_hbm, prefix_hbm, suffix_hbm, out_hbm, buf0, buf1, si0, si1, so0, so1):
        bufs = (buf0, buf1)
        sin = (si0, si1)
        sout = (so0, so1)
        wid = lax.axis_index("s") * info.num_cores + lax.axis_index("c")
        base = wid * cpw

        pltpu.sync_copy(ctx_hbm, buf0.at[pl.ds(d, ctx_sz)])
        pltpu.sync_copy(ctx_hbm, buf1.at[pl.ds(d, ctx_sz)])

        def cls(j):
            return jnp.minimum(base + j, n_cls - 1)

        in_descs = [None] * cpw
        out_descs = [None] * cpw

        def fire_in(j):
            p = j % 2
            i = cls(j)
            d1 = pltpu.async_copy(
                prefix_hbm.at[pl.ds(i * d, d)], bufs[p].at[pl.ds(0, d)], sin[p]
            )
            d2 = pltpu.async_copy(
                suffix_hbm.at[pl.ds(i * suf_sz, suf_sz)],
                bufs[p].at[pl.ds(d + ctx_sz, suf_sz)],
                sin[p],
            )
            in_descs[j] = (d1, d2)

        def fire_out(j):
            p = j % 2
            out_descs[j] = pltpu.async_copy(
                bufs[p], out_hbm.at[pl.ds(cls(j) * row, row)], sout[p]
            )

        fire_in(0)
        for j in range(cpw):
            if j >= 1:
                out_descs[j - 1].wait()
            if j + 1 < cpw:
                fire_in(j + 1)
            in_descs[j][0].wait()
            in_descs[j][1].wait()
            fire_out(j)
        out_descs[cpw - 1].wait()

    flat = body(ctx.reshape(-1), prefix.reshape(-1), suffix.reshape(-1))
    return flat.reshape(n_cls, seq, d)
